# baseline (speedup 1.0000x reference)
import jax
import jax.numpy as jnp
from jax.experimental import pallas as pl

_NUM_EXPERTS = 64
_TOP_K = 8
_BLOCK_T = 2048


def _gate_body(h_ref, w_ref, probs_ref, tkw_ref, tki_ref):
    h = h_ref[...]
    w = w_ref[...]
    logits_t = jax.lax.dot_general(
        w, h, (((1,), (1,)), ((), ())), preferred_element_type=jnp.float32
    )
    m = jnp.max(logits_t, axis=0, keepdims=True)
    e = jnp.exp(logits_t - m)
    s = jnp.sum(e, axis=0, keepdims=True)
    probs_t = e / s
    probs_ref[...] = probs_t.T

    iota = jax.lax.broadcasted_iota(jnp.int32, probs_t.shape, 0)
    work = probs_t
    w_rows = []
    i_rows = []
    for _ in range(_TOP_K):
        cur = jnp.max(work, axis=0, keepdims=True)
        idx = jnp.min(
            jnp.where(work == cur, iota, _NUM_EXPERTS), axis=0, keepdims=True
        )
        w_rows.append(cur)
        i_rows.append(idx)
        work = jnp.where(iota == idx, -1.0, work)
    tkw_ref[...] = jnp.concatenate(w_rows, axis=0).T
    tki_ref[...] = jnp.concatenate(i_rows, axis=0).T


def kernel(hidden_states, W_router):
    d_model = hidden_states.shape[-1]
    h2d = hidden_states.reshape(-1, d_model)
    n_tok = h2d.shape[0]
    grid = (n_tok // _BLOCK_T,)
    probs, tkw, tki = pl.pallas_call(
        _gate_body,
        grid=grid,
        in_specs=[
            pl.BlockSpec((_BLOCK_T, d_model), lambda i: (i, 0)),
            pl.BlockSpec((_NUM_EXPERTS, d_model), lambda i: (0, 0)),
        ],
        out_specs=[
            pl.BlockSpec((_BLOCK_T, _NUM_EXPERTS), lambda i: (i, 0)),
            pl.BlockSpec((_BLOCK_T, _TOP_K), lambda i: (i, 0)),
            pl.BlockSpec((_BLOCK_T, _TOP_K), lambda i: (i, 0)),
        ],
        out_shape=[
            jax.ShapeDtypeStruct((n_tok, _NUM_EXPERTS), jnp.float32),
            jax.ShapeDtypeStruct((n_tok, _TOP_K), jnp.float32),
            jax.ShapeDtypeStruct((n_tok, _TOP_K), jnp.int32),
        ],
    )(h2d, W_router)
    return (probs, tkw, tki)
